# baseline (speedup 1.0000x reference)
import functools

import jax
import jax.numpy as jnp
from jax import lax
from jax.experimental import pallas as pl
from jax.experimental.pallas import tpu as pltpu
from jax.experimental.pallas import tpu_sc as plsc

B = 16384
USER_DIM = 32
ITEM_DIM = 64
HIDDEN = 64
MARGIN = 1.0
LANES = 128

NC = 2
NS = 16
NW = NC * NS
BPW = B // NW
CHUNK = 128
NCHUNK = BPW // CHUNK


def _sc_gather_wide(uidx, pidx, nidx, utab_w, itab_w):
    mesh = plsc.VectorSubcoreMesh(core_axis_name="c", subcore_axis_name="s")

    @functools.partial(
        pl.kernel,
        out_type=[
            jax.ShapeDtypeStruct((B, LANES), jnp.float32),
            jax.ShapeDtypeStruct((B, LANES), jnp.float32),
            jax.ShapeDtypeStruct((B, LANES), jnp.float32),
        ],
        mesh=mesh,
        scratch_types=[
            pltpu.VMEM((BPW,), jnp.int32),
            pltpu.VMEM((BPW,), jnp.int32),
            pltpu.VMEM((BPW,), jnp.int32),
            pltpu.VMEM((2, CHUNK, LANES), jnp.float32),
            pltpu.VMEM((2, CHUNK, LANES), jnp.float32),
            pltpu.VMEM((2, CHUNK, LANES), jnp.float32),
            pltpu.SemaphoreType.DMA,
            pltpu.SemaphoreType.DMA,
            pltpu.SemaphoreType.DMA,
        ],
    )
    def k(uidx_hbm, pidx_hbm, nidx_hbm, utab_hbm, itab_hbm,
          ue_hbm, pe_hbm, ne_hbm,
          uidx_v, pidx_v, nidx_v, ubuf, pbuf, nbuf, su, sp, sn):
        wid = lax.axis_index("s") * NC + lax.axis_index("c")
        base = wid * BPW

        pltpu.sync_copy(uidx_hbm.at[wid], uidx_v)
        pltpu.sync_copy(pidx_hbm.at[wid], pidx_v)
        pltpu.sync_copy(nidx_hbm.at[wid], nidx_v)

        prev = None
        for j in range(NCHUNK):
            cur = j & 1
            sl = pl.ds(j * CHUNK, CHUNK)
            w = [
                pltpu.async_copy(utab_hbm.at[uidx_v.at[sl]], ubuf.at[cur], su),
                pltpu.async_copy(itab_hbm.at[pidx_v.at[sl]], pbuf.at[cur], sp),
                pltpu.async_copy(itab_hbm.at[nidx_v.at[sl]], nbuf.at[cur], sn),
            ]
            if prev is not None:
                pj, pw = prev
                for h in pw:
                    h.wait()
                dsl = pl.ds(base + pj * CHUNK, CHUNK)
                pltpu.sync_copy(ubuf.at[pj & 1], ue_hbm.at[dsl])
                pltpu.sync_copy(pbuf.at[pj & 1], pe_hbm.at[dsl])
                pltpu.sync_copy(nbuf.at[pj & 1], ne_hbm.at[dsl])
            prev = (j, w)
        pj, pw = prev
        for h in pw:
            h.wait()
        dsl = pl.ds(base + pj * CHUNK, CHUNK)
        pltpu.sync_copy(ubuf.at[pj & 1], ue_hbm.at[dsl])
        pltpu.sync_copy(pbuf.at[pj & 1], pe_hbm.at[dsl])
        pltpu.sync_copy(nbuf.at[pj & 1], ne_hbm.at[dsl])

    return k(uidx, pidx, nidx, utab_w, itab_w)


BLK = 4096


def _mlp_loss_kernel(uw_ref, pw_ref, nw_ref, ui_ref, pi_ref, ni_ref,
                     w1u_ref, w1i_ref, b1_ref, w2t_ref, b2_ref, out_ref):
    uw = uw_ref[...]
    ui = ui_ref[...]
    ub0 = (ui & 1) == 1
    ub1 = (ui & 2) == 2
    lo = jnp.where(ub0, uw[:, 32:64], uw[:, 0:32])
    hi = jnp.where(ub0, uw[:, 96:128], uw[:, 64:96])
    ue = jnp.where(ub1, hi, lo)

    pw = pw_ref[...]
    pe = jnp.where((pi_ref[...] & 1) == 1, pw[:, 64:128], pw[:, 0:64])
    nw = nw_ref[...]
    ne = jnp.where((ni_ref[...] & 1) == 1, nw[:, 64:128], nw[:, 0:64])

    u = jnp.dot(ue, w1u_ref[...], preferred_element_type=jnp.float32)
    b1 = b1_ref[...]
    hp = jnp.maximum(
        u + jnp.dot(pe, w1i_ref[...], preferred_element_type=jnp.float32)
        + b1, 0.0)
    hn = jnp.maximum(
        u + jnp.dot(ne, w1i_ref[...], preferred_element_type=jnp.float32)
        + b1, 0.0)
    w2t = w2t_ref[...]
    b2 = b2_ref[0, 0]
    op = jnp.maximum(jnp.sum(hp * w2t, axis=1, keepdims=True) + b2, 0.0)
    on = jnp.maximum(jnp.sum(hn * w2t, axis=1, keepdims=True) + b2, 0.0)
    part = jnp.sum(jnp.maximum(on - op + MARGIN, 0.0)) / B

    @pl.when(pl.program_id(0) == 0)
    def _():
        out_ref[0, 0] = 0.0
    out_ref[0, 0] += part


def kernel(user_input, pos_item_input, neg_item_input, user_table,
           item_table, W1, b1, W2, b2):
    n_users, n_items = user_table.shape[0], item_table.shape[0]
    utab_w = user_table.reshape(n_users // 4, LANES)
    itab_w = item_table.reshape(n_items // 2, LANES)

    uidx_w = (user_input >> 2).reshape(NW, BPW)
    pidx_w = (pos_item_input >> 1).reshape(NW, BPW)
    nidx_w = (neg_item_input >> 1).reshape(NW, BPW)

    ue_w, pe_w, ne_w = _sc_gather_wide(uidx_w, pidx_w, nidx_w, utab_w, itab_w)

    w1u = W1[:USER_DIM]
    w1i = W1[USER_DIM:]
    b1r = b1.reshape(1, HIDDEN)
    w2t = W2.reshape(1, HIDDEN)
    b2r = b2.reshape(1, 1)

    grid = B // BLK
    bspec_wide = pl.BlockSpec((BLK, LANES), lambda i: (i, 0))
    bspec_idx = pl.BlockSpec((BLK, 1), lambda i: (i, 0))
    bspec_full = lambda shape: pl.BlockSpec(shape, lambda i: (0, 0))

    loss = pl.pallas_call(
        _mlp_loss_kernel,
        grid=(grid,),
        in_specs=[
            bspec_wide, bspec_wide, bspec_wide,
            bspec_idx, bspec_idx, bspec_idx,
            bspec_full((USER_DIM, HIDDEN)),
            bspec_full((ITEM_DIM, HIDDEN)),
            bspec_full((1, HIDDEN)),
            bspec_full((1, HIDDEN)),
            bspec_full((1, 1)),
        ],
        out_specs=pl.BlockSpec(memory_space=pltpu.SMEM,
                               block_shape=(1, 1), index_map=lambda i: (0, 0)),
        out_shape=jax.ShapeDtypeStruct((1, 1), jnp.float32),
    )(ue_w, pe_w, ne_w,
      user_input.reshape(B, 1), pos_item_input.reshape(B, 1),
      neg_item_input.reshape(B, 1),
      w1u, w1i, b1r, w2t, b2r)
    return loss[0, 0]


# speedup vs baseline: 1.7457x; 1.7457x over previous
import functools

import jax
import jax.numpy as jnp
from jax import lax
from jax.experimental import pallas as pl
from jax.experimental.pallas import tpu as pltpu
from jax.experimental.pallas import tpu_sc as plsc

B = 16384
USER_DIM = 32
ITEM_DIM = 64
HIDDEN = 64
MARGIN = 1.0
LANES = 128

H_U = 1 << 18
H_I = 1 << 19

NC = 2
NS = 16
NW = NC * NS
BPW = B // NW
CHUNK = 128
NCHUNK = BPW // CHUNK

TBLK = 4096


def _user_transpose_kernel(t0_ref, t1_ref, t2_ref, t3_ref, out_ref):
    eye = jnp.eye(USER_DIM, dtype=jnp.float32)
    dn = (((0,), (0,)), ((), ()))
    parts = [
        lax.dot_general(r[...], eye, dn, preferred_element_type=jnp.float32)
        for r in (t0_ref, t1_ref, t2_ref, t3_ref)
    ]
    out_ref[...] = jnp.concatenate(parts, axis=1)


def _item_project_kernel(lo_ref, hi_ref, w_ref, out_ref):
    dn = (((0,), (0,)), ((), ()))
    w = w_ref[...]
    plo = lax.dot_general(lo_ref[...], w, dn,
                          preferred_element_type=jnp.float32)
    phi = lax.dot_general(hi_ref[...], w, dn,
                          preferred_element_type=jnp.float32)
    out_ref[...] = jnp.concatenate([plo, phi], axis=1)


def _sc_gather_wide(uidx, pidx, nidx, utab_w, itab_w):
    mesh = plsc.VectorSubcoreMesh(core_axis_name="c", subcore_axis_name="s")

    @functools.partial(
        pl.kernel,
        out_type=[
            jax.ShapeDtypeStruct((B, LANES), jnp.float32),
            jax.ShapeDtypeStruct((B, LANES), jnp.float32),
            jax.ShapeDtypeStruct((B, LANES), jnp.float32),
        ],
        mesh=mesh,
        scratch_types=[
            pltpu.VMEM((8, CHUNK), jnp.int32),
            pltpu.VMEM((8, CHUNK), jnp.int32),
            pltpu.VMEM((8, CHUNK), jnp.int32),
            pltpu.VMEM((2, CHUNK, LANES), jnp.float32),
            pltpu.VMEM((2, CHUNK, LANES), jnp.float32),
            pltpu.VMEM((2, CHUNK, LANES), jnp.float32),
            pltpu.SemaphoreType.DMA,
            pltpu.SemaphoreType.DMA,
            pltpu.SemaphoreType.DMA,
        ],
    )
    def k(uidx_hbm, pidx_hbm, nidx_hbm, utab_hbm, itab_hbm,
          ue_hbm, pe_hbm, ne_hbm,
          uidx_v, pidx_v, nidx_v, ubuf, pbuf, nbuf, su, sp, sn):
        wid = lax.axis_index("s") * NC + lax.axis_index("c")
        base = wid * BPW

        pltpu.sync_copy(uidx_hbm.at[wid], uidx_v)
        pltpu.sync_copy(pidx_hbm.at[wid], pidx_v)
        pltpu.sync_copy(nidx_hbm.at[wid], nidx_v)

        def store(j):
            dsl = pl.ds(base + j * CHUNK, CHUNK)
            pltpu.sync_copy(ubuf.at[j & 1], ue_hbm.at[dsl])
            pltpu.sync_copy(pbuf.at[j & 1], pe_hbm.at[dsl])
            pltpu.sync_copy(nbuf.at[j & 1], ne_hbm.at[dsl])

        prev = None
        for j in range(NCHUNK):
            cur = j & 1
            w = [
                pltpu.async_copy(utab_hbm.at[uidx_v.at[j]], ubuf.at[cur], su),
                pltpu.async_copy(itab_hbm.at[pidx_v.at[j]], pbuf.at[cur], sp),
                pltpu.async_copy(itab_hbm.at[nidx_v.at[j]], nbuf.at[cur], sn),
            ]
            if prev is not None:
                for h in prev[1]:
                    h.wait()
                store(prev[0])
            prev = (j, w)
        for h in prev[1]:
            h.wait()
        store(prev[0])

    return k(uidx, pidx, nidx, utab_w, itab_w)


MBLK = 4096


def _mlp_loss_kernel(uw_ref, pw_ref, nw_ref, ui_ref, pi_ref, ni_ref,
                     w1u_ref, b1_ref, w2t_ref, b2_ref, out_ref):
    uw = uw_ref[...]
    ui = ui_ref[...]
    ub0 = (ui & (1 << 18)) > 0
    ub1 = (ui & (1 << 19)) > 0
    lo = jnp.where(ub0, uw[:, 32:64], uw[:, 0:32])
    hi = jnp.where(ub0, uw[:, 96:128], uw[:, 64:96])
    ue = jnp.where(ub1, hi, lo)

    pw = pw_ref[...]
    pP = jnp.where(pi_ref[...] >= H_I, pw[:, 64:128], pw[:, 0:64])
    nw = nw_ref[...]
    nP = jnp.where(ni_ref[...] >= H_I, nw[:, 64:128], nw[:, 0:64])

    u1 = jnp.dot(ue, w1u_ref[...], preferred_element_type=jnp.float32)
    b1 = b1_ref[...]
    hp = jnp.maximum(u1 + pP + b1, 0.0)
    hn = jnp.maximum(u1 + nP + b1, 0.0)
    w2t = w2t_ref[...]
    b2 = b2_ref[0, 0]
    op = jnp.maximum(jnp.sum(hp * w2t, axis=1, keepdims=True) + b2, 0.0)
    on = jnp.maximum(jnp.sum(hn * w2t, axis=1, keepdims=True) + b2, 0.0)
    part = jnp.sum(jnp.maximum(on - op + MARGIN, 0.0)) / B

    @pl.when(pl.program_id(0) == 0)
    def _():
        out_ref[0, 0] = 0.0
    out_ref[0, 0] += part


def kernel(user_input, pos_item_input, neg_item_input, user_table,
           item_table, W1, b1, W2, b2):
    ut_u = user_table.T
    ut_i = item_table.T

    n_users, n_items = user_table.shape[0], item_table.shape[0]
    ulast = (n_users - 1) // TBLK
    ilast = (n_items - 1) // TBLK

    uw_tab = pl.pallas_call(
        _user_transpose_kernel,
        grid=(H_U // TBLK,),
        in_specs=[
            pl.BlockSpec((USER_DIM, TBLK),
                         lambda i, q=q: (0, jnp.minimum(
                             i + q * (H_U // TBLK), ulast)))
            for q in range(4)
        ],
        out_specs=pl.BlockSpec((TBLK, LANES), lambda i: (i, 0)),
        out_shape=jax.ShapeDtypeStruct((H_U, LANES), jnp.float32),
    )(ut_u, ut_u, ut_u, ut_u)

    w1i = W1[USER_DIM:]
    iw_tab = pl.pallas_call(
        _item_project_kernel,
        grid=(H_I // TBLK,),
        in_specs=[
            pl.BlockSpec((ITEM_DIM, TBLK), lambda i: (0, i)),
            pl.BlockSpec((ITEM_DIM, TBLK),
                         lambda i: (0, jnp.minimum(i + H_I // TBLK, ilast))),
            pl.BlockSpec((ITEM_DIM, HIDDEN), lambda i: (0, 0)),
        ],
        out_specs=pl.BlockSpec((TBLK, LANES), lambda i: (i, 0)),
        out_shape=jax.ShapeDtypeStruct((H_I, LANES), jnp.float32),
    )(ut_i, ut_i, w1i)

    idx3 = lambda a: jnp.pad(a.reshape(NW, NCHUNK, CHUNK),
                             ((0, 0), (0, 8 - NCHUNK), (0, 0)))
    uidx = idx3(user_input & (H_U - 1))
    pidx = idx3(pos_item_input & (H_I - 1))
    nidx = idx3(neg_item_input & (H_I - 1))

    ue_w, pe_w, ne_w = _sc_gather_wide(uidx, pidx, nidx, uw_tab, iw_tab)

    w1u = W1[:USER_DIM]
    b1r = b1.reshape(1, HIDDEN)
    w2t = W2.reshape(1, HIDDEN)
    b2r = b2.reshape(1, 1)

    grid = B // MBLK
    bspec_wide = pl.BlockSpec((MBLK, LANES), lambda i: (i, 0))
    bspec_idx = pl.BlockSpec((MBLK, 1), lambda i: (i, 0))
    bspec_full = lambda shape: pl.BlockSpec(shape, lambda i: (0, 0))

    loss = pl.pallas_call(
        _mlp_loss_kernel,
        grid=(grid,),
        in_specs=[
            bspec_wide, bspec_wide, bspec_wide,
            bspec_idx, bspec_idx, bspec_idx,
            bspec_full((USER_DIM, HIDDEN)),
            bspec_full((1, HIDDEN)),
            bspec_full((1, HIDDEN)),
            bspec_full((1, 1)),
        ],
        out_specs=pl.BlockSpec(memory_space=pltpu.SMEM,
                               block_shape=(1, 1), index_map=lambda i: (0, 0)),
        out_shape=jax.ShapeDtypeStruct((1, 1), jnp.float32),
    )(ue_w, pe_w, ne_w,
      user_input.reshape(B, 1), pos_item_input.reshape(B, 1),
      neg_item_input.reshape(B, 1),
      w1u, b1r, w2t, b2r)
    return loss[0, 0]


# speedup vs baseline: 2.2306x; 1.2778x over previous
import functools

import jax
import jax.numpy as jnp
from jax import lax
from jax.experimental import pallas as pl
from jax.experimental.pallas import tpu as pltpu
from jax.experimental.pallas import tpu_sc as plsc

B = 16384
USER_DIM = 32
ITEM_DIM = 64
HIDDEN = 64
MARGIN = 1.0
LANES = 128

HW = 1 << 18

NC = 2
NS = 16
NW = NC * NS
BPW = B // NW
CHUNK = 128
NCHUNK = BPW // CHUNK

TBLK = 4096


def _bf16_bits(x):
    u = lax.bitcast_convert_type(x, jnp.uint32)
    u = u + jnp.uint32(0x7FFF) + ((u >> 16) & jnp.uint32(1))
    return u


def _project_kernel(t0_ref, t1_ref, t2_ref, t3_ref, w_ref, out_ref):
    dn = (((0,), (0,)), ((), ()))
    w = w_ref[...]
    q = [lax.dot_general(r[...], w, dn, preferred_element_type=jnp.float32)
         for r in (t0_ref, t1_ref, t2_ref, t3_ref)]
    b = [_bf16_bits(x) for x in q]
    p01 = (b[0] >> 16) | (b[1] & jnp.uint32(0xFFFF0000))
    p23 = (b[2] >> 16) | (b[3] & jnp.uint32(0xFFFF0000))
    packed = jnp.concatenate([p01, p23], axis=1)
    out_ref[...] = lax.bitcast_convert_type(packed, jnp.float32)


def _make_wide(table_t, w_part, in_dim, last_block):
    return pl.pallas_call(
        _project_kernel,
        grid=(HW // TBLK,),
        in_specs=[
            pl.BlockSpec((in_dim, TBLK),
                         lambda i, q=q: (0, jnp.minimum(
                             i + q * (HW // TBLK), last_block)))
            for q in range(4)
        ] + [pl.BlockSpec((in_dim, HIDDEN), lambda i: (0, 0))],
        out_specs=pl.BlockSpec((TBLK, LANES), lambda i: (i, 0)),
        out_shape=jax.ShapeDtypeStruct((HW, LANES), jnp.float32),
    )(table_t, table_t, table_t, table_t, w_part)


_MESH = dict(core_axis_name="c", subcore_axis_name="s")


def _sc_gather1(idx, tab):
    @functools.partial(
        pl.kernel,
        out_type=jax.ShapeDtypeStruct((B, LANES), jnp.float32),
        mesh=plsc.VectorSubcoreMesh(**_MESH),
        scratch_types=[
            pltpu.VMEM((8, CHUNK), jnp.int32),
            pltpu.VMEM((2, CHUNK, LANES), jnp.float32),
            pltpu.SemaphoreType.DMA,
        ],
    )
    def k(idx_hbm, tab_hbm, out_hbm, idx_v, buf, sem):
        wid = lax.axis_index("s") * NC + lax.axis_index("c")
        base = wid * BPW
        pltpu.sync_copy(idx_hbm.at[wid], idx_v)
        prev = None
        for j in range(NCHUNK):
            h = pltpu.async_copy(tab_hbm.at[idx_v.at[j]], buf.at[j & 1], sem)
            if prev is not None:
                prev[1].wait()
                pltpu.sync_copy(buf.at[prev[0] & 1],
                                out_hbm.at[pl.ds(base + prev[0] * CHUNK, CHUNK)])
            prev = (j, h)
        prev[1].wait()
        pltpu.sync_copy(buf.at[prev[0] & 1],
                        out_hbm.at[pl.ds(base + prev[0] * CHUNK, CHUNK)])

    return k(idx, tab)


def _sc_gather2(pidx, nidx, tab):
    @functools.partial(
        pl.kernel,
        out_type=[jax.ShapeDtypeStruct((B, LANES), jnp.float32),
                  jax.ShapeDtypeStruct((B, LANES), jnp.float32)],
        mesh=plsc.VectorSubcoreMesh(**_MESH),
        scratch_types=[
            pltpu.VMEM((8, CHUNK), jnp.int32),
            pltpu.VMEM((8, CHUNK), jnp.int32),
            pltpu.VMEM((2, CHUNK, LANES), jnp.float32),
            pltpu.VMEM((2, CHUNK, LANES), jnp.float32),
            pltpu.SemaphoreType.DMA,
            pltpu.SemaphoreType.DMA,
        ],
    )
    def k(pidx_hbm, nidx_hbm, tab_hbm, pe_hbm, ne_hbm,
          pidx_v, nidx_v, pbuf, nbuf, sp, sn):
        wid = lax.axis_index("s") * NC + lax.axis_index("c")
        base = wid * BPW
        pltpu.sync_copy(pidx_hbm.at[wid], pidx_v)
        pltpu.sync_copy(nidx_hbm.at[wid], nidx_v)

        def store(j):
            dsl = pl.ds(base + j * CHUNK, CHUNK)
            pltpu.sync_copy(pbuf.at[j & 1], pe_hbm.at[dsl])
            pltpu.sync_copy(nbuf.at[j & 1], ne_hbm.at[dsl])

        prev = None
        for j in range(NCHUNK):
            w = [pltpu.async_copy(tab_hbm.at[pidx_v.at[j]], pbuf.at[j & 1], sp),
                 pltpu.async_copy(tab_hbm.at[nidx_v.at[j]], nbuf.at[j & 1], sn)]
            if prev is not None:
                for h in prev[1]:
                    h.wait()
                store(prev[0])
            prev = (j, w)
        for h in prev[1]:
            h.wait()
        store(prev[0])

    return k(pidx, nidx, tab)


MBLK = 4096
_U32 = jnp.uint32


def _unpack(x, idx, bit_half, bit_par):
    half = jnp.where((idx & _U32(bit_half)) > 0, x[:, 64:128], x[:, 0:64])
    bits = lax.bitcast_convert_type(half, jnp.uint32)
    lo = lax.bitcast_convert_type(bits << 16, jnp.float32)
    hi = lax.bitcast_convert_type(bits & _U32(0xFFFF0000), jnp.float32)
    return jnp.where((idx & _U32(bit_par)) > 0, hi, lo)


def _mlp_loss_kernel(uw_ref, pw_ref, nw_ref, ui_ref, pi_ref, ni_ref,
                     b1_ref, w2t_ref, b2_ref, out_ref):
    ui = ui_ref[...].astype(jnp.uint32)
    pi = pi_ref[...].astype(jnp.uint32)
    ni = ni_ref[...].astype(jnp.uint32)
    u1 = _unpack(uw_ref[...], ui, 1 << 19, 1 << 18)
    pP = _unpack(pw_ref[...], pi, 1 << 19, 1 << 18)
    nP = _unpack(nw_ref[...], ni, 1 << 19, 1 << 18)

    b1 = b1_ref[...]
    hp = jnp.maximum(u1 + pP + b1, 0.0)
    hn = jnp.maximum(u1 + nP + b1, 0.0)
    w2t = w2t_ref[...]
    b2 = b2_ref[0, 0]
    op = jnp.maximum(jnp.sum(hp * w2t, axis=1, keepdims=True) + b2, 0.0)
    on = jnp.maximum(jnp.sum(hn * w2t, axis=1, keepdims=True) + b2, 0.0)
    part = jnp.sum(jnp.maximum(on - op + MARGIN, 0.0)) / B

    @pl.when(pl.program_id(0) == 0)
    def _():
        out_ref[0, 0] = 0.0
    out_ref[0, 0] += part


def kernel(user_input, pos_item_input, neg_item_input, user_table,
           item_table, W1, b1, W2, b2):
    ut_u = user_table.T
    ut_i = item_table.T
    n_users, n_items = user_table.shape[0], item_table.shape[0]
    ulast = (n_users - 1) // TBLK
    ilast = (n_items - 1) // TBLK

    uw_tab = _make_wide(ut_u, W1[:USER_DIM], USER_DIM, ulast)
    iw_tab = _make_wide(ut_i, W1[USER_DIM:], ITEM_DIM, ilast)

    idx3 = lambda a: jnp.pad((a & (HW - 1)).reshape(NW, NCHUNK, CHUNK),
                             ((0, 0), (0, 8 - NCHUNK), (0, 0)))
    ue_w = _sc_gather1(idx3(user_input), uw_tab)
    pe_w, ne_w = _sc_gather2(idx3(pos_item_input), idx3(neg_item_input),
                             iw_tab)

    b1r = b1.reshape(1, HIDDEN)
    w2t = W2.reshape(1, HIDDEN)
    b2r = b2.reshape(1, 1)

    grid = B // MBLK
    bspec_wide = pl.BlockSpec((MBLK, LANES), lambda i: (i, 0))
    bspec_idx = pl.BlockSpec((MBLK, 1), lambda i: (i, 0))
    bspec_full = lambda shape: pl.BlockSpec(shape, lambda i: (0, 0))

    loss = pl.pallas_call(
        _mlp_loss_kernel,
        grid=(grid,),
        in_specs=[
            bspec_wide, bspec_wide, bspec_wide,
            bspec_idx, bspec_idx, bspec_idx,
            bspec_full((1, HIDDEN)),
            bspec_full((1, HIDDEN)),
            bspec_full((1, 1)),
        ],
        out_specs=pl.BlockSpec(memory_space=pltpu.SMEM,
                               block_shape=(1, 1), index_map=lambda i: (0, 0)),
        out_shape=jax.ShapeDtypeStruct((1, 1), jnp.float32),
    )(ue_w, pe_w, ne_w,
      user_input.reshape(B, 1), pos_item_input.reshape(B, 1),
      neg_item_input.reshape(B, 1),
      b1r, w2t, b2r)
    return loss[0, 0]
